# baseline (speedup 1.0000x reference)
import jax
import jax.numpy as jnp
from jax.experimental import pallas as pl
from jax.experimental.pallas import tpu as pltpu

VOCAB_ = 1000000
DIMS_ = 64
BLK = 65536
NB = (VOCAB_ + BLK - 1) // BLK
OBLK = 4 * BLK
NOB = (VOCAB_ + OBLK - 1) // OBLK
NEG_INF = float("-inf")


def _body(idx_ref, table_ref, w_ref, b_ref, out_ref, z_ref, m_ref, s_ref,
          e_ref):
    t = pl.program_id(0)

    @pl.when(t == 0)
    def _init():
        m_ref[...] = jnp.full_like(m_ref, NEG_INF)
        s_ref[...] = jnp.zeros_like(s_ref)
        lane = idx_ref[0] % 128
        tb = table_ref[...]
        li = jax.lax.broadcasted_iota(jnp.int32, (DIMS_, 128), 1)
        e_ref[...] = jnp.sum(jnp.where(li == lane, tb, 0.0), axis=1,
                             keepdims=True)

    def _accum(zm):
        bm = jnp.max(zm, axis=1, keepdims=True)
        new_m = jnp.maximum(m_ref[...], bm)
        s_ref[...] = s_ref[...] * jnp.exp(m_ref[...] - new_m) + jnp.sum(
            jnp.exp(zm - new_m), axis=1, keepdims=True)
        m_ref[...] = new_m

    @pl.when(t < NB)
    def _compute():
        z = jax.lax.dot_general(
            e_ref[...], w_ref[...], (((0,), (0,)), ((), ())),
            preferred_element_type=jnp.float32)
        z = z + b_ref[...]
        z_ref[:, pl.ds(t * BLK, BLK)] = z

        @pl.when(t < NB - 1)
        def _full():
            _accum(z)

        @pl.when(t == NB - 1)
        def _partial():
            col = t * BLK + jax.lax.broadcasted_iota(jnp.int32, (1, BLK), 1)
            _accum(jnp.where(col < VOCAB_, z, NEG_INF))

    @pl.when(t >= NB)
    def _write():
        j = t - NB
        lse = m_ref[...] + jnp.log(s_ref[...])
        out_ref[...] = z_ref[:, pl.ds(j * OBLK, OBLK)] - lse


@jax.jit
def _run(inputs, tableT, WT, b2):
    grid_spec = pltpu.PrefetchScalarGridSpec(
        num_scalar_prefetch=1,
        grid=(NB + NOB,),
        in_specs=[
            pl.BlockSpec((DIMS_, 128), lambda t, idx: (0, idx[0] // 128)),
            pl.BlockSpec((DIMS_, BLK), lambda t, idx: (0, jnp.minimum(t, NB - 1))),
            pl.BlockSpec((1, BLK), lambda t, idx: (0, jnp.minimum(t, NB - 1))),
        ],
        out_specs=pl.BlockSpec(
            (1, OBLK), lambda t, idx: (0, jnp.where(t < NB, 0, t - NB))),
        scratch_shapes=[
            pltpu.VMEM((1, NB * BLK), jnp.float32),
            pltpu.VMEM((1, 1), jnp.float32),
            pltpu.VMEM((1, 1), jnp.float32),
            pltpu.VMEM((DIMS_, 1), jnp.float32),
        ],
    )
    return pl.pallas_call(
        _body,
        grid_spec=grid_spec,
        out_shape=jax.ShapeDtypeStruct((1, VOCAB_), jnp.float32),
        compiler_params=pltpu.CompilerParams(
            dimension_semantics=("arbitrary",),
        ),
    )(inputs, tableT, WT, b2)


def kernel(inputs, table, W, b):
    idx = inputs.astype(jnp.int32)
    return _run(idx, table.T, W.T, b.reshape(1, VOCAB_))


# speedup vs baseline: 1.0192x; 1.0192x over previous
import jax
import jax.numpy as jnp
from jax.experimental import pallas as pl
from jax.experimental.pallas import tpu as pltpu

VOCAB_ = 1000000
DIMS_ = 64
BLK = 32768
NB = (VOCAB_ + BLK - 1) // BLK
OBLK = 8 * BLK
NOB = (VOCAB_ + OBLK - 1) // OBLK
NEG_INF = float("-inf")


def _body(idx_ref, table_ref, w_ref, b_ref, out_ref, z_ref, m_ref, s_ref,
          e_ref):
    t = pl.program_id(0)

    @pl.when(t == 0)
    def _init():
        m_ref[...] = jnp.full_like(m_ref, NEG_INF)
        s_ref[...] = jnp.zeros_like(s_ref)
        lane = idx_ref[0] % 128
        tb = table_ref[...]
        li = jax.lax.broadcasted_iota(jnp.int32, (DIMS_, 128), 1)
        e_ref[...] = jnp.sum(jnp.where(li == lane, tb, 0.0), axis=1,
                             keepdims=True)

    def _accum(zm):
        bm = jnp.max(zm, axis=1, keepdims=True)
        new_m = jnp.maximum(m_ref[...], bm)
        s_ref[...] = s_ref[...] * jnp.exp(m_ref[...] - new_m) + jnp.sum(
            jnp.exp(zm - new_m), axis=1, keepdims=True)
        m_ref[...] = new_m

    @pl.when(t < NB)
    def _compute():
        z = jax.lax.dot_general(
            e_ref[...], w_ref[...], (((0,), (0,)), ((), ())),
            preferred_element_type=jnp.float32)
        z = z + b_ref[...]
        z_ref[:, pl.ds(t * BLK, BLK)] = z

        @pl.when(t < NB - 1)
        def _full():
            _accum(z)

        @pl.when(t == NB - 1)
        def _partial():
            col = t * BLK + jax.lax.broadcasted_iota(jnp.int32, (1, BLK), 1)
            _accum(jnp.where(col < VOCAB_, z, NEG_INF))

    @pl.when(t >= NB)
    def _write():
        j = t - NB
        lse = m_ref[...] + jnp.log(s_ref[...])
        out_ref[...] = z_ref[:, pl.ds(j * OBLK, OBLK)] - lse


@jax.jit
def _run(inputs, tableT, WT, b2):
    grid_spec = pltpu.PrefetchScalarGridSpec(
        num_scalar_prefetch=1,
        grid=(NB + NOB,),
        in_specs=[
            pl.BlockSpec((DIMS_, 128), lambda t, idx: (0, idx[0] // 128)),
            pl.BlockSpec((DIMS_, BLK), lambda t, idx: (0, jnp.minimum(t, NB - 1))),
            pl.BlockSpec((1, BLK), lambda t, idx: (0, jnp.minimum(t, NB - 1))),
        ],
        out_specs=pl.BlockSpec(
            (1, OBLK), lambda t, idx: (0, jnp.where(t < NB, 0, t - NB))),
        scratch_shapes=[
            pltpu.VMEM((1, NB * BLK), jnp.float32),
            pltpu.VMEM((1, 1), jnp.float32),
            pltpu.VMEM((1, 1), jnp.float32),
            pltpu.VMEM((DIMS_, 1), jnp.float32),
        ],
    )
    return pl.pallas_call(
        _body,
        grid_spec=grid_spec,
        out_shape=jax.ShapeDtypeStruct((1, VOCAB_), jnp.float32),
        compiler_params=pltpu.CompilerParams(
            dimension_semantics=("arbitrary",),
        ),
    )(inputs, tableT, WT, b2)


def kernel(inputs, table, W, b):
    idx = inputs.astype(jnp.int32)
    return _run(idx, table.T, W.T, b.reshape(1, VOCAB_))
